# baseline (speedup 1.0000x reference)
import jax
import jax.numpy as jnp
from jax.experimental import pallas as pl
from jax.experimental.pallas import tpu as pltpu

_CHUNK = 640000
_NSTEPS = 10


_HALF = _CHUNK // 2


def _start(b_hbm, scratch, sems, k, step, slot):
    base = step * _CHUNK
    pltpu.make_async_copy(
        b_hbm.at[k, pl.ds(base, _HALF)],
        scratch.at[slot, pl.ds(0, _HALF)], sems.at[slot, 0],
    ).start()
    pltpu.make_async_copy(
        b_hbm.at[k, pl.ds(base + _HALF, _HALF)],
        scratch.at[slot, pl.ds(_HALF, _HALF)], sems.at[slot, 1],
    ).start()


def _wait(b_hbm, scratch, sems, k, step, slot):
    base = step * _CHUNK
    pltpu.make_async_copy(
        b_hbm.at[k, pl.ds(base, _HALF)],
        scratch.at[slot, pl.ds(0, _HALF)], sems.at[slot, 0],
    ).wait()
    pltpu.make_async_copy(
        b_hbm.at[k, pl.ds(base + _HALF, _HALF)],
        scratch.at[slot, pl.ds(_HALF, _HALF)], sems.at[slot, 1],
    ).wait()


def _body(k_ref, b_hbm, o_ref, scratch, sems):
    i = pl.program_id(0)
    k = k_ref[0]
    slot = jax.lax.rem(i, 2)
    nxt = jax.lax.rem(i + 1, 2)

    @pl.when(i == 0)
    def _first():
        _start(b_hbm, scratch, sems, k, i, slot)

    @pl.when(i + 1 < _NSTEPS)
    def _prefetch():
        _start(b_hbm, scratch, sems, k, i + 1, nxt)

    _wait(b_hbm, scratch, sems, k, i, slot)
    o_ref[...] = jax.nn.sigmoid(scratch[slot])


def kernel(B, k, edge_index, n):
    E = B.shape[1]
    k_arr = jnp.atleast_1d(k).astype(jnp.int32)
    out = pl.pallas_call(
        _body,
        grid_spec=pltpu.PrefetchScalarGridSpec(
            num_scalar_prefetch=1,
            grid=(_NSTEPS,),
            in_specs=[pl.BlockSpec(memory_space=pl.ANY)],
            out_specs=pl.BlockSpec((_CHUNK,), lambda i, kref: (i,)),
            scratch_shapes=[
                pltpu.VMEM((2, _CHUNK), jnp.float32),
                pltpu.SemaphoreType.DMA((2, 2)),
            ],
        ),
        out_shape=jax.ShapeDtypeStruct((E,), jnp.float32),
    )(k_arr, B)
    return out


# speedup vs baseline: 1.1888x; 1.1888x over previous
import jax
import jax.numpy as jnp
from jax.experimental import pallas as pl
from jax.experimental.pallas import tpu as pltpu

_CHUNK = 640000
_NSTEPS = 10


def _body(k_ref, b_hbm, o_ref, scratch, sems):
    i = pl.program_id(0)
    k = k_ref[0]

    @pl.when(i == 0)
    def _enqueue_all():
        for j in range(_NSTEPS):
            pltpu.make_async_copy(
                b_hbm.at[k, pl.ds(j * _CHUNK, _CHUNK)],
                scratch.at[j], sems.at[j],
            ).start()

    pltpu.make_async_copy(
        b_hbm.at[k, pl.ds(i * _CHUNK, _CHUNK)], scratch.at[i], sems.at[i]
    ).wait()
    o_ref[...] = jax.nn.sigmoid(scratch[i])


def kernel(B, k, edge_index, n):
    E = B.shape[1]
    k_arr = jnp.atleast_1d(k).astype(jnp.int32)
    out = pl.pallas_call(
        _body,
        grid_spec=pltpu.PrefetchScalarGridSpec(
            num_scalar_prefetch=1,
            grid=(_NSTEPS,),
            in_specs=[pl.BlockSpec(memory_space=pl.ANY)],
            out_specs=pl.BlockSpec((_CHUNK,), lambda i, kref: (i,)),
            scratch_shapes=[
                pltpu.VMEM((_NSTEPS, _CHUNK), jnp.float32),
                pltpu.SemaphoreType.DMA((_NSTEPS,)),
            ],
        ),
        out_shape=jax.ShapeDtypeStruct((E,), jnp.float32),
    )(k_arr, B)
    return out
